# baseline (speedup 1.0000x reference)
import functools

import jax
import jax.numpy as jnp
from jax import lax
from jax.experimental import pallas as pl
from jax.experimental.pallas import tpu as pltpu
from jax.experimental.pallas import tpu_sc as plsc

B, IN, N, C, TAU = 4096, 1024, 8192, 16, 10.0
NC, NS, LANES = 2, 16, 16
NW = NC * NS
NPW = N // NW
G = 8


def _compute_coeffs(w_v, c0_v, ca_v, cb_v, cab_v):
    def group(q, carry):
        sl = pl.ds(q * LANES, LANES)
        rows = [w_v[g, sl] for g in range(16)]
        m = rows[0]
        for g in range(1, 16):
            m = jnp.maximum(m, rows[g])
        e = [jnp.exp(r - m) for r in rows]
        s = e[0]
        for g in range(1, 16):
            s = s + e[g]
        inv = 1.0 / s
        c0r = ((e[8] + e[9]) + (e[10] + e[11])) + ((e[12] + e[13]) + (e[14] + e[15]))
        car = ((e[2] + e[3]) + (e[6] + e[7])) - ((e[8] + e[9]) + (e[12] + e[13]))
        cbr = ((e[4] + e[5]) + (e[6] + e[7])) - ((e[8] + e[9]) + (e[10] + e[11]))
        cabr = (((e[1] + e[8]) + 2.0 * e[9]) + (e[11] + e[13])) - \
               (((e[2] + e[4]) + 2.0 * e[6]) + (e[7] + e[14]))
        c0_v[sl] = c0r * inv
        ca_v[sl] = car * inv
        cb_v[sl] = cbr * inv
        cab_v[sl] = cabr * inv
        return carry

    lax.fori_loop(0, NPW // LANES, group, 0)


def _make_layer(in_dim, final):
    mesh = plsc.VectorSubcoreMesh(core_axis_name="c", subcore_axis_name="s")
    if final:
        out_t = jax.ShapeDtypeStruct((2, C, B), jnp.float32)
        o_scratch = pltpu.VMEM((B,), jnp.float32)
    else:
        out_t = jax.ShapeDtypeStruct((N, B), jnp.float32)
        o_scratch = pltpu.VMEM((G, B), jnp.float32)

    @functools.partial(
        pl.kernel,
        mesh=mesh,
        out_type=out_t,
        scratch_types=[
            pltpu.VMEM((NPW,), jnp.int32),
            pltpu.VMEM((NPW,), jnp.int32),
            pltpu.VMEM((16, NPW), jnp.float32),
            pltpu.VMEM((NPW + LANES,), jnp.float32),
            pltpu.VMEM((NPW + LANES,), jnp.float32),
            pltpu.VMEM((NPW + LANES,), jnp.float32),
            pltpu.VMEM((NPW + LANES,), jnp.float32),
            pltpu.VMEM((G, B), jnp.float32),
            pltpu.VMEM((G, B), jnp.float32),
            o_scratch,
            pltpu.SemaphoreType.DMA,
            pltpu.SemaphoreType.DMA,
        ],
    )
    def layer(h_hbm, ia_hbm, ib_hbm, wt_hbm, out_hbm,
              ia_v, ib_v, w_v, c0_v, ca_v, cb_v, cab_v,
              a_v, b_v, o_v, sem_a, sem_b):
        wid = lax.axis_index("s") * NC + lax.axis_index("c")
        base = wid * NPW
        pltpu.sync_copy(ia_hbm.at[pl.ds(base, NPW)], ia_v)
        pltpu.sync_copy(ib_hbm.at[pl.ds(base, NPW)], ib_v)
        pltpu.sync_copy(wt_hbm.at[:, pl.ds(base, NPW)], w_v)
        _compute_coeffs(w_v, c0_v, ca_v, cb_v, cab_v)

        if final:
            def zero_body(t, carry):
                sl = pl.ds(t * LANES, LANES)
                o_v[sl] = o_v[sl] * 0.0
                return carry
            lax.fori_loop(0, B // LANES, zero_body, 0)

        def chunk(g, carry):
            cp_a = pltpu.async_copy(h_hbm.at[ia_v.at[pl.ds(g * G, G)]], a_v, sem_a)
            cp_b = pltpu.async_copy(h_hbm.at[ib_v.at[pl.ds(g * G, G)]], b_v, sem_b)
            cp_a.wait()
            cp_b.wait()
            cs = pl.ds(g * G, LANES)
            c0g = c0_v[cs]
            cag = ca_v[cs]
            cbg = cb_v[cs]
            cabg = cab_v[cs]
            for gg in range(G):
                c0 = c0g[gg]
                ca = cag[gg]
                cb = cbg[gg]
                cab = cabg[gg]

                def inner(t, icarry):
                    sl = pl.ds(t * LANES, LANES)
                    a = a_v[gg, sl]
                    b = b_v[gg, sl]
                    r = c0 + ca * a + cb * b + cab * (a * b)
                    if final:
                        plsc.addupdate(o_v.at[sl], r)
                    else:
                        o_v[gg, sl] = r
                    return icarry

                lax.fori_loop(0, B // LANES, inner, 0)
            if not final:
                pltpu.sync_copy(o_v, out_hbm.at[pl.ds(base + g * G, G)])
            return carry

        lax.fori_loop(0, NPW // G, chunk, 0)
        if final:
            pltpu.sync_copy(o_v, out_hbm.at[wid % 2, wid // 2])

    return layer


_layer0 = _make_layer(IN, False)
_layer_mid = _make_layer(N, False)
_layer_last = _make_layer(N, True)


def _combine_body(p_ref, o_ref):
    o_ref[...] = (p_ref[0] + p_ref[1]) * (1.0 / TAU)


def _combine(part):
    blk = 512
    return pl.pallas_call(
        _combine_body,
        grid=(B // blk,),
        in_specs=[pl.BlockSpec((2, C, blk), lambda i: (0, 0, i))],
        out_specs=pl.BlockSpec((C, blk), lambda i: (0, i)),
        out_shape=jax.ShapeDtypeStruct((C, B), jnp.float32),
    )(part)


def kernel(x, idx_a0, idx_b0, w0, idx_a1, idx_b1, w1,
           idx_a2, idx_b2, w2, idx_a3, idx_b3, w3):
    h = x.T
    h = _layer0(h, idx_a0, idx_b0, w0.T)
    h = _layer_mid(h, idx_a1, idx_b1, w1.T)
    h = _layer_mid(h, idx_a2, idx_b2, w2.T)
    part = _layer_last(h, idx_a3, idx_b3, w3.T)
    return _combine(part).T
